# baseline (speedup 1.0000x reference)
import functools

import jax
import jax.numpy as jnp
from jax import lax
from jax.experimental import pallas as pl
from jax.experimental.pallas import tpu as pltpu
from jax.experimental.pallas import tpu_sc as plsc

_NC = 2
_NS = 16
_NW = _NC * _NS
_BL = 128


@functools.lru_cache(maxsize=None)
def _build(b_sz: int, hist: int, vocab: int, d: int, nbuf: int):
    assert b_sz == _NW * _BL and d % 8 == 0
    n_dr = d // 8
    assert hist % nbuf == 0 and hist // nbuf >= 2

    mesh = plsc.VectorSubcoreMesh(
        core_axis_name="c", subcore_axis_name="s",
        num_cores=_NC, num_subcores=_NS)

    @functools.partial(
        pl.kernel,
        out_type=jax.ShapeDtypeStruct((hist, n_dr, _NW, 8, _BL), jnp.float32),
        mesh=mesh,
        scratch_types=[
            pltpu.VMEM((hist, _BL), jnp.int32),
            pltpu.VMEM((nbuf, _BL, d), jnp.float32),
            pltpu.VMEM((nbuf, n_dr, 8, _BL), jnp.float32),
            pltpu.SemaphoreType.DMA((nbuf,)),
            pltpu.SemaphoreType.DMA((nbuf,)),
        ],
        compiler_params=pltpu.CompilerParams(
            use_tc_tiling_on_sc=False, needs_layout_passes=False),
    )
    def k(idxt_hbm, table_hbm, out_hbm, idx_v, g_v, t_v, gsem, ssem):
        wid = lax.axis_index("s") * _NC + lax.axis_index("c")

        pltpu.sync_copy(idxt_hbm.at[:, pl.ds(wid * _BL, _BL)], idx_v)

        lanes = jnp.arange(16, dtype=jnp.int32)
        rows = [lanes + 16 * j for j in range(_BL // 16)]

        def gfire(h, b):
            pltpu.async_copy(
                table_hbm.at[idx_v.at[h]],
                g_v.at[b], gsem.at[b])

        def gdrain(b):
            pltpu.make_async_copy(
                table_hbm.at[pl.ds(0, _BL)],
                g_v.at[b], gsem.at[b]).wait()

        def transpose(b):
            @pl.loop(0, d, unroll=4)
            def _dd(dd):
                dr = dd // 8
                ds_ = dd % 8
                col = jnp.full((16,), dd, dtype=jnp.int32)
                vs = [plsc.load_gather(g_v.at[b], [r, col]) for r in rows]
                for j, v in enumerate(vs):
                    t_v.at[b, dr, ds_][pl.ds(16 * j, 16)] = v

        def sfire(h, b):
            pltpu.async_copy(
                t_v.at[b], out_hbm.at[h, :, wid], ssem.at[b])

        def sdrain(b):
            pltpu.make_async_copy(
                t_v.at[b], out_hbm.at[0, :, 0], ssem.at[b]).wait()

        n_rounds = hist // nbuf

        @pl.loop(0, n_rounds + 2)
        def _round(k):
            for b in range(nbuf):
                @pl.when(jnp.logical_and(k >= 1, k <= n_rounds))
                def _retire():
                    gdrain(b)

                @pl.when(jnp.logical_and(k >= 2, k <= n_rounds + 1))
                def _free():
                    sdrain(b)

                @pl.when(jnp.logical_and(k >= 1, k <= n_rounds))
                def _emit():
                    transpose(b)
                    sfire((k - 1) * nbuf + b, b)

                @pl.when(k < n_rounds)
                def _fire():
                    gfire(k * nbuf + b, b)

    return k


def kernel(input_seqs, table):
    batch, hist = input_seqs.shape
    vocab, d = table.shape
    idxt = input_seqs.astype(jnp.int32).T
    o5 = _build(batch, hist, vocab, d, 8)(idxt, table)
    return o5.transpose(2, 4, 0, 1, 3).reshape(batch, hist, d)


# speedup vs baseline: 1.3947x; 1.3947x over previous
import functools

import jax
import jax.numpy as jnp
from jax import lax
from jax.experimental import pallas as pl
from jax.experimental.pallas import tpu as pltpu
from jax.experimental.pallas import tpu_sc as plsc

_NC = 2
_NS = 16
_NW = _NC * _NS
_BL = 128


@functools.lru_cache(maxsize=None)
def _build(b_sz: int, hist: int, vocab: int, d: int, nbuf: int):
    assert b_sz == _NW * _BL and d % 8 == 0 and (d & (d - 1)) == 0
    n_dr = d // 8
    assert hist % nbuf == 0 and hist // nbuf >= 2

    mesh = plsc.VectorSubcoreMesh(
        core_axis_name="c", subcore_axis_name="s",
        num_cores=_NC, num_subcores=_NS)

    @functools.partial(
        pl.kernel,
        out_type=jax.ShapeDtypeStruct((hist * n_dr * _NW * 8 * _BL,), jnp.float32),
        mesh=mesh,
        scratch_types=[
            pltpu.VMEM((hist, _BL), jnp.int32),
            pltpu.VMEM((nbuf, _BL, d), jnp.float32),
            pltpu.VMEM((nbuf, 1, d * _BL), jnp.float32),
            pltpu.SemaphoreType.DMA((nbuf,)),
            pltpu.SemaphoreType.DMA((nbuf,)),
        ],
        compiler_params=pltpu.CompilerParams(
            use_tc_tiling_on_sc=False, needs_layout_passes=False),
    )
    def k(idxt_hbm, table_hbm, out_hbm, idx_v, g_v, t_v, gsem, ssem):
        wid = lax.axis_index("s") * _NC + lax.axis_index("c")

        pltpu.sync_copy(idxt_hbm.at[:, pl.ds(wid * _BL, _BL)], idx_v)

        lanes = jnp.arange(16, dtype=jnp.int32)
        rows = [lanes + 16 * j for j in range(_BL // 16)]

        def gfire(h, b):
            pltpu.async_copy(
                table_hbm.at[idx_v.at[h]],
                g_v.at[b], gsem.at[b])

        def gdrain(b):
            pltpu.make_async_copy(
                table_hbm.at[pl.ds(0, _BL)],
                g_v.at[b], gsem.at[b]).wait()

        def transpose(b):
            @pl.loop(0, d, unroll=4)
            def _dd(dd):
                col = (dd + lanes) & (d - 1)
                tbase = col * _BL
                for r in rows:
                    v = plsc.load_gather(g_v.at[b], [r, col])
                    plsc.store_scatter(t_v.at[b, 0], [tbase + r], v)

        def sfire(h, b):
            for dr in range(n_dr):
                pltpu.async_copy(
                    t_v.at[b, 0].at[pl.ds(dr * 8 * _BL, 8 * _BL)],
                    out_hbm.at[pl.ds(((h * n_dr + dr) * _NW + wid) * 8 * _BL,
                                     8 * _BL)],
                    ssem.at[b])

        def sdrain(b):
            pltpu.make_async_copy(
                t_v.at[b, 0], out_hbm.at[pl.ds(0, d * _BL)],
                ssem.at[b]).wait()

        n_rounds = hist // nbuf

        @pl.loop(0, n_rounds + 2)
        def _round(k):
            for b in range(nbuf):
                @pl.when(jnp.logical_and(k >= 1, k <= n_rounds))
                def _retire():
                    gdrain(b)

                @pl.when(jnp.logical_and(k >= 2, k <= n_rounds + 1))
                def _free():
                    sdrain(b)

                @pl.when(jnp.logical_and(k >= 1, k <= n_rounds))
                def _emit():
                    transpose(b)
                    sfire((k - 1) * nbuf + b, b)

                @pl.when(k < n_rounds)
                def _fire():
                    gfire(k * nbuf + b, b)

    return k


def kernel(input_seqs, table):
    batch, hist = input_seqs.shape
    vocab, d = table.shape
    idxt = input_seqs.astype(jnp.int32).T
    of = _build(batch, hist, vocab, d, 8)(idxt, table)
    o5 = of.reshape(hist, d // 8, _NW, 8, _BL)
    return o5.transpose(2, 4, 0, 1, 3).reshape(batch, hist, d)


# speedup vs baseline: 2.0061x; 1.4384x over previous
import functools

import jax
import jax.numpy as jnp
from jax import lax
from jax.experimental import pallas as pl
from jax.experimental.pallas import tpu as pltpu
from jax.experimental.pallas import tpu_sc as plsc

_NC = 2
_NS = 16
_NW = _NC * _NS
_BL = 128


@functools.lru_cache(maxsize=None)
def _build(b_sz: int, hist: int, vocab: int, d: int, nbuf: int):
    assert b_sz == _NW * _BL and d % 8 == 0 and (d & (d - 1)) == 0
    n_dr = d // 8
    assert hist % nbuf == 0 and hist // nbuf >= 2

    mesh = plsc.VectorSubcoreMesh(
        core_axis_name="c", subcore_axis_name="s",
        num_cores=_NC, num_subcores=_NS)

    @functools.partial(
        pl.kernel,
        out_type=jax.ShapeDtypeStruct((hist * n_dr * _NW * 8 * _BL,), jnp.float32),
        mesh=mesh,
        scratch_types=[
            pltpu.VMEM((hist, _BL), jnp.int32),
            pltpu.VMEM((nbuf, _BL, d), jnp.float32),
            pltpu.VMEM((nbuf, 1, d * _BL), jnp.float32),
            pltpu.SemaphoreType.DMA((nbuf,)),
            pltpu.SemaphoreType.DMA((nbuf,)),
        ],
        compiler_params=pltpu.CompilerParams(
            use_tc_tiling_on_sc=False, needs_layout_passes=False),
    )
    def k(idxt_hbm, table_hbm, out_hbm, idx_v, g_v, t_v, gsem, ssem):
        wid = lax.axis_index("s") * _NC + lax.axis_index("c")

        pltpu.sync_copy(idxt_hbm.at[:, pl.ds(wid * _BL, _BL)], idx_v)

        lanes = jnp.arange(16, dtype=jnp.int32)
        rows = [lanes + 16 * j for j in range(_BL // 16)]

        def gfire(h, b):
            pltpu.async_copy(
                table_hbm.at[idx_v.at[h]],
                g_v.at[b], gsem.at[b])

        def gdrain(b):
            pltpu.make_async_copy(
                table_hbm.at[pl.ds(0, _BL)],
                g_v.at[b], gsem.at[b]).wait()

        def transpose(b):
            @pl.loop(0, d, unroll=4)
            def _dd(dd):
                col = (dd + lanes) & (d - 1)
                tbase = col * _BL
                for r in rows:
                    v = plsc.load_gather(g_v.at[b], [r, col])
                    plsc.store_scatter(t_v.at[b, 0], [tbase + r], v)

        def sfire(h, b):
            for dr in range(n_dr):
                pltpu.async_copy(
                    t_v.at[b, 0].at[pl.ds(dr * 8 * _BL, 8 * _BL)],
                    out_hbm.at[pl.ds(((h * n_dr + dr) * _NW + wid) * 8 * _BL,
                                     8 * _BL)],
                    ssem.at[b])

        def sdrain(b):
            pltpu.make_async_copy(
                t_v.at[b, 0], out_hbm.at[pl.ds(0, d * _BL)],
                ssem.at[b]).wait()

        n_rounds = hist // nbuf

        @pl.loop(0, n_rounds + 2)
        def _round(k):
            for b in range(nbuf):
                @pl.when(jnp.logical_and(k >= 1, k <= n_rounds))
                def _retire():
                    gdrain(b)

                @pl.when(jnp.logical_and(k >= 2, k <= n_rounds + 1))
                def _free():
                    sdrain(b)

                @pl.when(jnp.logical_and(k >= 1, k <= n_rounds))
                def _emit():
                    transpose(b)
                    sfire((k - 1) * nbuf + b, b)

                @pl.when(k < n_rounds)
                def _fire():
                    gfire(k * nbuf + b, b)

    return k


@functools.lru_cache(maxsize=None)
def _build_detranspose(vocab: int, d: int):
    assert d == 32 and vocab % 8 == 0
    ncol_full = vocab // 128
    tail = vocab - ncol_full * 128
    n_cols = ncol_full + (1 if tail else 0)
    per_w, rem = divmod(ncol_full, _NW)

    mesh = plsc.VectorSubcoreMesh(
        core_axis_name="c", subcore_axis_name="s",
        num_cores=_NC, num_subcores=_NS)

    @functools.partial(
        pl.kernel,
        out_type=jax.ShapeDtypeStruct((n_cols * 128 * d,), jnp.float32),
        mesh=mesh,
        scratch_types=[
            pltpu.VMEM((2, d, 128), jnp.float32),
            pltpu.VMEM((2, 1, d * 128), jnp.float32),
            pltpu.SemaphoreType.DMA((2,)),
            pltpu.SemaphoreType.DMA((2,)),
        ],
        compiler_params=pltpu.CompilerParams(
            use_tc_tiling_on_sc=True, needs_layout_passes=False),
    )
    def kt(tt_hbm, tailpad_hbm, out_hbm, s_v, r_v, isem, osem):
        wid = lax.axis_index("s") * _NC + lax.axis_index("c")
        lanes = jnp.arange(16, dtype=jnp.int32)
        start = wid * per_w + jnp.minimum(wid, rem)
        ncol_w = per_w + (wid < rem).astype(jnp.int32)

        def ifire(c, b):
            pltpu.async_copy(
                tt_hbm.at[:, pl.ds((start + c) * 128, 128)],
                s_v.at[b], isem.at[b])

        def idrain(b):
            pltpu.make_async_copy(
                tt_hbm.at[:, pl.ds(0, 128)], s_v.at[b], isem.at[b]).wait()

        def transpose(b):
            @pl.loop(0, d, unroll=4)
            def _d0(d0):
                dv = (d0 + lanes) & (d - 1)
                for j in range(8):
                    rl = lanes + 16 * j
                    v = plsc.load_gather(s_v.at[b], [dv, rl])
                    plsc.store_scatter(r_v.at[b, 0], [rl * d + dv], v)

        def ofire(c, b):
            pltpu.async_copy(
                r_v.at[b, 0],
                out_hbm.at[pl.ds((start + c) * 128 * d, 128 * d)],
                osem.at[b])

        def odrain(b):
            pltpu.make_async_copy(
                r_v.at[b, 0], out_hbm.at[pl.ds(0, 128 * d)],
                osem.at[b]).wait()

        @pl.loop(0, per_w + 3)
        def _ring(k):
            for b in range(2):
                c_fire = 2 * k + b
                c_ret = 2 * (k - 1) + b
                c_pp = 2 * (k - 2) + b

                @pl.when(jnp.logical_and(c_ret >= 0, c_ret < ncol_w))
                def _ret():
                    idrain(b)

                @pl.when(jnp.logical_and(c_pp >= 0, c_pp < ncol_w))
                def _fr():
                    odrain(b)

                @pl.when(jnp.logical_and(c_ret >= 0, c_ret < ncol_w))
                def _em():
                    transpose(b)
                    ofire(c_ret, b)

                @pl.when(c_fire < ncol_w)
                def _fi():
                    ifire(c_fire, b)

        if tail:
            @pl.when(wid == 0)
            def _tail():
                pltpu.sync_copy(tailpad_hbm, s_v.at[0])
                transpose(0)
                pltpu.sync_copy(
                    r_v.at[0, 0],
                    out_hbm.at[pl.ds(ncol_full * 128 * d, 128 * d)])

    return kt


def kernel(input_seqs, table):
    batch, hist = input_seqs.shape
    vocab, d = table.shape
    idxt = input_seqs.astype(jnp.int32).T
    tt = table.T
    ncol_full = vocab // 128
    tail = vocab - ncol_full * 128
    tailpad = jnp.pad(lax.slice(tt, (0, ncol_full * 128), (d, vocab)),
                      ((0, 0), (0, 128 - tail)))
    tflat = _build_detranspose(vocab, d)(tt, tailpad)
    vpad = (ncol_full + (1 if tail else 0)) * 128
    table_lin = tflat.reshape(vpad, d)
    of = _build(batch, hist, vpad, d, 8)(idxt, table_lin)
    o5 = of.reshape(hist, d // 8, _NW, 8, _BL)
    return o5.transpose(2, 4, 0, 1, 3).reshape(batch, hist, d)
